# baseline (speedup 1.0000x reference)
import functools

import jax
import jax.numpy as jnp
from jax import lax
from jax.experimental import pallas as pl
from jax.experimental.pallas import tpu as pltpu
from jax.experimental.pallas import tpu_sc as plsc

B = 16384
D = 64
NC = 2
NS = 16
NW = NC * NS
BPW = B // NW


RPC = BPW // 2


def _sc_gather_body(uq_hbm, eq_hbm, uo_hbm, eo_hbm,
                    uemb_hbm, eemb_hbm, ub_hbm, eb_hbm,
                    ue_out, ee_out, ub_out, eb_out,
                    uidx_v, eidx_v, uoidx_v, eoidx_v,
                    urows_v, erows_v, ubv, ebv,
                    semu, seme, semb):
    wid = lax.axis_index("s") * NC + lax.axis_index("c")
    base = wid * BPW
    pltpu.sync_copy(uq_hbm.at[wid], uidx_v)
    pltpu.sync_copy(eq_hbm.at[wid], eidx_v)
    pltpu.sync_copy(uo_hbm.at[wid], uoidx_v)
    pltpu.sync_copy(eo_hbm.at[wid], eoidx_v)
    for h in range(2):
        cu = pltpu.async_copy(uemb_hbm.at[uidx_v.at[h]], urows_v, semu)
        ce = pltpu.async_copy(eemb_hbm.at[eidx_v.at[h]], erows_v, seme)
        cub = pltpu.async_copy(ub_hbm.at[uoidx_v.at[h]], ubv, semb)
        ceb = pltpu.async_copy(eb_hbm.at[eoidx_v.at[h]], ebv, semb)
        cu.wait()
        pltpu.sync_copy(urows_v, ue_out.at[pl.ds(base + h * RPC, RPC)])
        ce.wait()
        pltpu.sync_copy(erows_v, ee_out.at[pl.ds(base + h * RPC, RPC)])
        cub.wait()
        ceb.wait()
        pltpu.sync_copy(ubv, ub_out.at[pl.ds(base + h * RPC, RPC)])
        pltpu.sync_copy(ebv, eb_out.at[pl.ds(base + h * RPC, RPC)])


@functools.cache
def _sc_gather():
    return pl.kernel(
        _sc_gather_body,
        out_type=[
            jax.ShapeDtypeStruct((B, 2 * D), jnp.float32),
            jax.ShapeDtypeStruct((B, 2 * D), jnp.float32),
            jax.ShapeDtypeStruct((B, 1), jnp.float32),
            jax.ShapeDtypeStruct((B, 1), jnp.float32),
        ],
        mesh=plsc.VectorSubcoreMesh(core_axis_name="c", subcore_axis_name="s"),
        compiler_params=pltpu.CompilerParams(use_tc_tiling_on_sc=False),
        scratch_types=[
            pltpu.VMEM((2, RPC), jnp.int32),
            pltpu.VMEM((2, RPC), jnp.int32),
            pltpu.VMEM((2, RPC), jnp.int32),
            pltpu.VMEM((2, RPC), jnp.int32),
            pltpu.VMEM((RPC, 2 * D), jnp.float32),
            pltpu.VMEM((RPC, 2 * D), jnp.float32),
            pltpu.VMEM((RPC, 1), jnp.float32),
            pltpu.VMEM((RPC, 1), jnp.float32),
            pltpu.SemaphoreType.DMA,
            pltpu.SemaphoreType.DMA,
            pltpu.SemaphoreType.DMA,
        ],
    )


BLK = 1024
NBLK = B // BLK
_EPS = 1e-5

_row_spec = lambda w: pl.BlockSpec((BLK, w), lambda t: (t, 0))
_full_spec = lambda r, c: pl.BlockSpec((r, c), lambda t: (0, 0))
_part_spec = pl.BlockSpec((1, 1, 256), lambda t: (t, 0, 0))
_part_spec128 = pl.BlockSpec((1, 1, 128), lambda t: (t, 0, 0))


def _pick_half(two_rows, par):
    return jnp.where(par == 1, two_rows[:, D:], two_rows[:, :D])


def _phase1_body(ue2_ref, up_ref, ee2_ref, ep_ref, w1a_ref, w1b_ref, b1_ref,
                 h1_ref, ps_ref, pq_ref):
    ue = _pick_half(ue2_ref[...], up_ref[...])
    ee = _pick_half(ee2_ref[...], ep_ref[...])
    h = (jnp.dot(ue, w1a_ref[...], preferred_element_type=jnp.float32)
         + jnp.dot(ee, w1b_ref[...], preferred_element_type=jnp.float32)
         + b1_ref[...])
    h = jnp.maximum(h, 0.0)
    h1_ref[...] = h
    ps_ref[...] = jnp.sum(h, axis=0, keepdims=True).reshape(1, 1, 256)
    pq_ref[...] = jnp.sum(h * h, axis=0, keepdims=True).reshape(1, 1, 256)


_phase1 = pl.pallas_call(
    _phase1_body,
    grid=(NBLK,),
    in_specs=[_row_spec(2 * D), _row_spec(1), _row_spec(2 * D), _row_spec(1),
              _full_spec(D, 256), _full_spec(D, 256), _full_spec(1, 256)],
    out_specs=[_row_spec(256), _part_spec, _part_spec],
    out_shape=[
        jax.ShapeDtypeStruct((B, 256), jnp.float32),
        jax.ShapeDtypeStruct((NBLK, 1, 256), jnp.float32),
        jax.ShapeDtypeStruct((NBLK, 1, 256), jnp.float32),
    ],
)


def _phase2_body(h1_ref, sc_ref, sh_ref, w2_ref, b2_ref,
                 h2_ref, ps_ref, pq_ref):
    h = h1_ref[...] * sc_ref[...] + sh_ref[...]
    h = jnp.maximum(jnp.dot(h, w2_ref[...], preferred_element_type=jnp.float32)
                    + b2_ref[...], 0.0)
    h2_ref[...] = h
    ps_ref[...] = jnp.sum(h, axis=0, keepdims=True).reshape(1, 1, 128)
    pq_ref[...] = jnp.sum(h * h, axis=0, keepdims=True).reshape(1, 1, 128)


_phase2 = pl.pallas_call(
    _phase2_body,
    grid=(NBLK,),
    in_specs=[_row_spec(256), _full_spec(1, 256), _full_spec(1, 256),
              _full_spec(256, 128), _full_spec(1, 128)],
    out_specs=[_row_spec(128), _part_spec128, _part_spec128],
    out_shape=[
        jax.ShapeDtypeStruct((B, 128), jnp.float32),
        jax.ShapeDtypeStruct((NBLK, 1, 128), jnp.float32),
        jax.ShapeDtypeStruct((NBLK, 1, 128), jnp.float32),
    ],
)


def _phase3_body(h2_ref, sc_ref, sh_ref, w3_ref, b3_ref, w4_ref,
                 ue2_ref, up_ref, ee2_ref, ep_ref, ub_ref, eb_ref,
                 b4gb_ref, out_ref):
    h = h2_ref[...] * sc_ref[...] + sh_ref[...]
    h = jnp.maximum(jnp.dot(h, w3_ref[...], preferred_element_type=jnp.float32)
                    + b3_ref[...], 0.0)
    mlp_out = jnp.sum(h * w4_ref[...], axis=1, keepdims=True)
    ue = _pick_half(ue2_ref[...], up_ref[...])
    ee = _pick_half(ee2_ref[...], ep_ref[...])
    mf = jnp.sum(ue * ee, axis=1, keepdims=True) + ub_ref[...] + eb_ref[...]
    out_ref[...] = jax.nn.sigmoid(mlp_out + 0.3 * mf + b4gb_ref[0, 0])


_phase3 = pl.pallas_call(
    _phase3_body,
    grid=(NBLK,),
    in_specs=[_row_spec(128), _full_spec(1, 128), _full_spec(1, 128),
              _full_spec(128, D), _full_spec(1, D), _full_spec(1, D),
              _row_spec(2 * D), _row_spec(1), _row_spec(2 * D), _row_spec(1),
              _row_spec(1), _row_spec(1), _full_spec(1, 1)],
    out_specs=_row_spec(1),
    out_shape=jax.ShapeDtypeStruct((B, 1), jnp.float32),
)


def _bn_coeffs(ps, pq, g, be):
    m = ps.sum(axis=0)[0] * (1.0 / B)
    v = pq.sum(axis=0)[0] * (1.0 / B) - m * m
    s = g * lax.rsqrt(v + _EPS)
    return s.reshape(1, -1), (be - m * s).reshape(1, -1)


def _mlp(ue2, up, ee2, ep, ub, eb, w1a, w1b, b1, g1, be1, w2, b2, g2, be2,
         w3, b3, w4, b4gb):
    h1, ps1, pq1 = _phase1(ue2, up, ee2, ep, w1a, w1b, b1.reshape(1, -1))
    sc1, sh1 = _bn_coeffs(ps1, pq1, g1, be1)
    h2, ps2, pq2 = _phase2(h1, sc1, sh1, w2, b2.reshape(1, -1))
    sc2, sh2 = _bn_coeffs(ps2, pq2, g2, be2)
    return _phase3(h2, sc2, sh2, w3, b3.reshape(1, -1), w4,
                   ue2, up, ee2, ep, ub, eb, b4gb)


def kernel(user_ids, exercise_ids, user_emb, ex_emb, user_b, ex_b, global_b,
           W1, b1, g1, be1, W2, b2, g2, be2, W3, b3, W4, b4):
    uid = user_ids.astype(jnp.int32)
    eid = exercise_ids.astype(jnp.int32)
    uq = (uid // 2).reshape(NW, 2, RPC)
    eq = (eid // 2).reshape(NW, 2, RPC)
    uo = uid.reshape(NW, 2, RPC)
    eo = eid.reshape(NW, 2, RPC)
    up = (uid & 1).reshape(B, 1)
    ep = (eid & 1).reshape(B, 1)
    uemb2 = user_emb.reshape(-1, 2 * D)
    eemb2 = ex_emb.reshape(-1, 2 * D)
    ue2, ee2, ub, eb = _sc_gather()(uq, eq, uo, eo, uemb2, eemb2,
                                    user_b, ex_b)
    w1a = W1[:, :D].T
    w1b = W1[:, D:].T
    b4gb = (0.7 * b4 + 0.3 * global_b).reshape(1, 1)
    return _mlp(ue2, up, ee2, ep, ub, eb, w1a, w1b, b1, g1, be1, W2.T, b2,
                g2, be2, W3.T, b3, W4.reshape(1, D) * 0.7, b4gb)


# speedup vs baseline: 2.1865x; 2.1865x over previous
import functools

import jax
import jax.numpy as jnp
from jax import lax
from jax.experimental import pallas as pl
from jax.experimental.pallas import tpu as pltpu
from jax.experimental.pallas import tpu_sc as plsc

B = 16384
D = 64
NC = 2
NS = 16
NW = NC * NS
BPW = B // NW


RPC = BPW // 2


def _sc_gather_body(uq_hbm, eq_hbm, uo_hbm, eo_hbm,
                    uemb_hbm, eemb_hbm, ub_hbm, eb_hbm,
                    ue_out, ee_out, ub_out, eb_out,
                    uidx_v, eidx_v, uoidx_v, eoidx_v,
                    urows_v, erows_v, ubv, ebv,
                    semu, seme, semb):
    wid = lax.axis_index("s") * NC + lax.axis_index("c")
    base = wid * BPW
    pltpu.sync_copy(uq_hbm.at[wid], uidx_v)
    pltpu.sync_copy(eq_hbm.at[wid], eidx_v)
    pltpu.sync_copy(uo_hbm.at[wid], uoidx_v)
    pltpu.sync_copy(eo_hbm.at[wid], eoidx_v)
    for h in range(2):
        cu = pltpu.async_copy(uemb_hbm.at[uidx_v.at[h]], urows_v, semu)
        ce = pltpu.async_copy(eemb_hbm.at[eidx_v.at[h]], erows_v, seme)
        cub = pltpu.async_copy(ub_hbm.at[uoidx_v.at[h]], ubv, semb)
        ceb = pltpu.async_copy(eb_hbm.at[eoidx_v.at[h]], ebv, semb)
        cu.wait()
        pltpu.sync_copy(urows_v, ue_out.at[pl.ds(base + h * RPC, RPC)])
        ce.wait()
        pltpu.sync_copy(erows_v, ee_out.at[pl.ds(base + h * RPC, RPC)])
        cub.wait()
        ceb.wait()
        pltpu.sync_copy(ubv, ub_out.at[pl.ds(base + h * RPC, RPC)])
        pltpu.sync_copy(ebv, eb_out.at[pl.ds(base + h * RPC, RPC)])


@functools.cache
def _sc_gather():
    return pl.kernel(
        _sc_gather_body,
        out_type=[
            jax.ShapeDtypeStruct((B, 2 * D), jnp.float32),
            jax.ShapeDtypeStruct((B, 2 * D), jnp.float32),
            jax.ShapeDtypeStruct((B,), jnp.float32),
            jax.ShapeDtypeStruct((B,), jnp.float32),
        ],
        mesh=plsc.VectorSubcoreMesh(core_axis_name="c", subcore_axis_name="s"),
        compiler_params=pltpu.CompilerParams(use_tc_tiling_on_sc=False),
        scratch_types=[
            pltpu.VMEM((2, RPC), jnp.int32),
            pltpu.VMEM((2, RPC), jnp.int32),
            pltpu.VMEM((2, RPC), jnp.int32),
            pltpu.VMEM((2, RPC), jnp.int32),
            pltpu.VMEM((RPC, 2 * D), jnp.float32),
            pltpu.VMEM((RPC, 2 * D), jnp.float32),
            pltpu.VMEM((RPC,), jnp.float32),
            pltpu.VMEM((RPC,), jnp.float32),
            pltpu.SemaphoreType.DMA,
            pltpu.SemaphoreType.DMA,
            pltpu.SemaphoreType.DMA,
        ],
    )


BLK = 1024
NBLK = B // BLK
_EPS = 1e-5

_row_spec = lambda w: pl.BlockSpec((BLK, w), lambda t: (t, 0))
_full_spec = lambda r, c: pl.BlockSpec((r, c), lambda t: (0, 0))
_part_spec = pl.BlockSpec((1, 1, 256), lambda t: (t, 0, 0))
_part_spec128 = pl.BlockSpec((1, 1, 128), lambda t: (t, 0, 0))


def _pick_half(two_rows, par):
    return jnp.where(par == 1, two_rows[:, D:], two_rows[:, :D])


def _phase1_body(ue2_ref, up_ref, ee2_ref, ep_ref, w1a_ref, w1b_ref, b1_ref,
                 h1_ref, ps_ref, pq_ref):
    ue = _pick_half(ue2_ref[...], up_ref[...])
    ee = _pick_half(ee2_ref[...], ep_ref[...])
    h = (jnp.dot(ue, w1a_ref[...], preferred_element_type=jnp.float32)
         + jnp.dot(ee, w1b_ref[...], preferred_element_type=jnp.float32)
         + b1_ref[...])
    h = jnp.maximum(h, 0.0)
    h1_ref[...] = h
    ps_ref[...] = jnp.sum(h, axis=0, keepdims=True).reshape(1, 1, 256)
    pq_ref[...] = jnp.sum(h * h, axis=0, keepdims=True).reshape(1, 1, 256)


_phase1 = pl.pallas_call(
    _phase1_body,
    grid=(NBLK,),
    in_specs=[_row_spec(2 * D), _row_spec(1), _row_spec(2 * D), _row_spec(1),
              _full_spec(D, 256), _full_spec(D, 256), _full_spec(1, 256)],
    out_specs=[_row_spec(256), _part_spec, _part_spec],
    out_shape=[
        jax.ShapeDtypeStruct((B, 256), jnp.float32),
        jax.ShapeDtypeStruct((NBLK, 1, 256), jnp.float32),
        jax.ShapeDtypeStruct((NBLK, 1, 256), jnp.float32),
    ],
)


def _phase2_body(h1_ref, sc_ref, sh_ref, w2_ref, b2_ref,
                 h2_ref, ps_ref, pq_ref):
    h = h1_ref[...] * sc_ref[...] + sh_ref[...]
    h = jnp.maximum(jnp.dot(h, w2_ref[...], preferred_element_type=jnp.float32)
                    + b2_ref[...], 0.0)
    h2_ref[...] = h
    ps_ref[...] = jnp.sum(h, axis=0, keepdims=True).reshape(1, 1, 128)
    pq_ref[...] = jnp.sum(h * h, axis=0, keepdims=True).reshape(1, 1, 128)


_phase2 = pl.pallas_call(
    _phase2_body,
    grid=(NBLK,),
    in_specs=[_row_spec(256), _full_spec(1, 256), _full_spec(1, 256),
              _full_spec(256, 128), _full_spec(1, 128)],
    out_specs=[_row_spec(128), _part_spec128, _part_spec128],
    out_shape=[
        jax.ShapeDtypeStruct((B, 128), jnp.float32),
        jax.ShapeDtypeStruct((NBLK, 1, 128), jnp.float32),
        jax.ShapeDtypeStruct((NBLK, 1, 128), jnp.float32),
    ],
)


def _phase3_body(h2_ref, sc_ref, sh_ref, w3_ref, b3_ref, w4_ref,
                 ue2_ref, up_ref, ee2_ref, ep_ref, ub_ref, eb_ref,
                 b4gb_ref, out_ref):
    h = h2_ref[...] * sc_ref[...] + sh_ref[...]
    h = jnp.maximum(jnp.dot(h, w3_ref[...], preferred_element_type=jnp.float32)
                    + b3_ref[...], 0.0)
    mlp_out = jnp.sum(h * w4_ref[...], axis=1, keepdims=True)
    ue = _pick_half(ue2_ref[...], up_ref[...])
    ee = _pick_half(ee2_ref[...], ep_ref[...])
    mf = jnp.sum(ue * ee, axis=1, keepdims=True) + ub_ref[...] + eb_ref[...]
    out_ref[...] = jax.nn.sigmoid(mlp_out + 0.3 * mf + b4gb_ref[0, 0])


_phase3 = pl.pallas_call(
    _phase3_body,
    grid=(NBLK,),
    in_specs=[_row_spec(128), _full_spec(1, 128), _full_spec(1, 128),
              _full_spec(128, D), _full_spec(1, D), _full_spec(1, D),
              _row_spec(2 * D), _row_spec(1), _row_spec(2 * D), _row_spec(1),
              _row_spec(1), _row_spec(1), _full_spec(1, 1)],
    out_specs=_row_spec(1),
    out_shape=jax.ShapeDtypeStruct((B, 1), jnp.float32),
)


def _bn_coeffs(ps, pq, g, be):
    m = ps.sum(axis=0)[0] * (1.0 / B)
    v = pq.sum(axis=0)[0] * (1.0 / B) - m * m
    s = g * lax.rsqrt(v + _EPS)
    return s.reshape(1, -1), (be - m * s).reshape(1, -1)


def _mlp(ue2, up, ee2, ep, ub, eb, w1a, w1b, b1, g1, be1, w2, b2, g2, be2,
         w3, b3, w4, b4gb):
    h1, ps1, pq1 = _phase1(ue2, up, ee2, ep, w1a, w1b, b1.reshape(1, -1))
    sc1, sh1 = _bn_coeffs(ps1, pq1, g1, be1)
    h2, ps2, pq2 = _phase2(h1, sc1, sh1, w2, b2.reshape(1, -1))
    sc2, sh2 = _bn_coeffs(ps2, pq2, g2, be2)
    return _phase3(h2, sc2, sh2, w3, b3.reshape(1, -1), w4,
                   ue2, up, ee2, ep, ub, eb, b4gb)


def kernel(user_ids, exercise_ids, user_emb, ex_emb, user_b, ex_b, global_b,
           W1, b1, g1, be1, W2, b2, g2, be2, W3, b3, W4, b4):
    uid = user_ids.astype(jnp.int32)
    eid = exercise_ids.astype(jnp.int32)
    uq = (uid // 2).reshape(NW, 2, RPC)
    eq = (eid // 2).reshape(NW, 2, RPC)
    uo = uid.reshape(NW, 2, RPC)
    eo = eid.reshape(NW, 2, RPC)
    up = (uid & 1).reshape(B, 1)
    ep = (eid & 1).reshape(B, 1)
    uemb2 = user_emb.reshape(-1, 2 * D)
    eemb2 = ex_emb.reshape(-1, 2 * D)
    ue2, ee2, ub, eb = _sc_gather()(uq, eq, uo, eo, uemb2, eemb2,
                                    user_b.reshape(-1), ex_b.reshape(-1))
    w1a = W1[:, :D].T
    w1b = W1[:, D:].T
    b4gb = (0.7 * b4 + 0.3 * global_b).reshape(1, 1)
    return _mlp(ue2, up, ee2, ep, ub.reshape(B, 1), eb.reshape(B, 1), w1a,
                w1b, b1, g1, be1, W2.T, b2, g2, be2, W3.T, b3,
                W4.reshape(1, D) * 0.7, b4gb)


# speedup vs baseline: 2.2372x; 1.0232x over previous
import functools

import jax
import jax.numpy as jnp
from jax import lax
from jax.experimental import pallas as pl
from jax.experimental.pallas import tpu as pltpu
from jax.experimental.pallas import tpu_sc as plsc

B = 16384
D = 64
NC = 2
NS = 16
NW = NC * NS
BPW = B // NW


def _sc_gather_body(uid_hbm, eid_hbm, uemb_hbm, eemb_hbm, ub_hbm, eb_hbm,
                    ue_out, ee_out, ub_out, eb_out,
                    uidx_v, eidx_v, urows_v, erows_v, ubv, ebv,
                    semu, seme, semb):
    wid = lax.axis_index("s") * NC + lax.axis_index("c")
    base = wid * BPW
    pltpu.sync_copy(uid_hbm.at[wid], uidx_v)
    pltpu.sync_copy(eid_hbm.at[wid], eidx_v)
    cu = pltpu.async_copy(uemb_hbm.at[uidx_v.at[0]], urows_v, semu)
    ce = pltpu.async_copy(eemb_hbm.at[eidx_v.at[0]], erows_v, seme)
    cub = pltpu.async_copy(ub_hbm.at[uidx_v.at[0]], ubv, semb)
    ceb = pltpu.async_copy(eb_hbm.at[eidx_v.at[0]], ebv, semb)
    cu.wait()
    pltpu.sync_copy(urows_v, ue_out.at[pl.ds(base, BPW)])
    ce.wait()
    pltpu.sync_copy(erows_v, ee_out.at[pl.ds(base, BPW)])
    cub.wait()
    ceb.wait()
    pltpu.sync_copy(ubv, ub_out.at[pl.ds(base, BPW)])
    pltpu.sync_copy(ebv, eb_out.at[pl.ds(base, BPW)])


@functools.cache
def _sc_gather():
    return pl.kernel(
        _sc_gather_body,
        out_type=[
            jax.ShapeDtypeStruct((B, D), jnp.float32),
            jax.ShapeDtypeStruct((B, D), jnp.float32),
            jax.ShapeDtypeStruct((B,), jnp.float32),
            jax.ShapeDtypeStruct((B,), jnp.float32),
        ],
        mesh=plsc.VectorSubcoreMesh(core_axis_name="c", subcore_axis_name="s"),
        compiler_params=pltpu.CompilerParams(use_tc_tiling_on_sc=False),
        scratch_types=[
            pltpu.VMEM((1, BPW), jnp.int32),
            pltpu.VMEM((1, BPW), jnp.int32),
            pltpu.VMEM((BPW, D), jnp.float32),
            pltpu.VMEM((BPW, D), jnp.float32),
            pltpu.VMEM((BPW,), jnp.float32),
            pltpu.VMEM((BPW,), jnp.float32),
            pltpu.SemaphoreType.DMA,
            pltpu.SemaphoreType.DMA,
            pltpu.SemaphoreType.DMA,
        ],
    )


BLK = 1024
NBLK = B // BLK
_EPS = 1e-5

_row_spec = lambda w: pl.BlockSpec((BLK, w), lambda t: (t, 0))
_full_spec = lambda r, c: pl.BlockSpec((r, c), lambda t: (0, 0))
_part_spec = pl.BlockSpec((1, 1, 256), lambda t: (t, 0, 0))
_part_spec128 = pl.BlockSpec((1, 1, 128), lambda t: (t, 0, 0))


def _phase1_body(ue_ref, ee_ref, w1a_ref, w1b_ref, b1_ref,
                 h1_ref, ps_ref, pq_ref):
    h = (jnp.dot(ue_ref[...], w1a_ref[...], preferred_element_type=jnp.float32)
         + jnp.dot(ee_ref[...], w1b_ref[...], preferred_element_type=jnp.float32)
         + b1_ref[...])
    h = jnp.maximum(h, 0.0)
    h1_ref[...] = h
    ps_ref[...] = jnp.sum(h, axis=0, keepdims=True).reshape(1, 1, 256)
    pq_ref[...] = jnp.sum(h * h, axis=0, keepdims=True).reshape(1, 1, 256)


_phase1 = pl.pallas_call(
    _phase1_body,
    grid=(NBLK,),
    in_specs=[_row_spec(D), _row_spec(D), _full_spec(D, 256),
              _full_spec(D, 256), _full_spec(1, 256)],
    out_specs=[_row_spec(256), _part_spec, _part_spec],
    out_shape=[
        jax.ShapeDtypeStruct((B, 256), jnp.float32),
        jax.ShapeDtypeStruct((NBLK, 1, 256), jnp.float32),
        jax.ShapeDtypeStruct((NBLK, 1, 256), jnp.float32),
    ],
)


def _phase2_body(h1_ref, sc_ref, sh_ref, w2_ref, b2_ref,
                 h2_ref, ps_ref, pq_ref):
    h = h1_ref[...] * sc_ref[...] + sh_ref[...]
    h = jnp.maximum(jnp.dot(h, w2_ref[...], preferred_element_type=jnp.float32)
                    + b2_ref[...], 0.0)
    h2_ref[...] = h
    ps_ref[...] = jnp.sum(h, axis=0, keepdims=True).reshape(1, 1, 128)
    pq_ref[...] = jnp.sum(h * h, axis=0, keepdims=True).reshape(1, 1, 128)


_phase2 = pl.pallas_call(
    _phase2_body,
    grid=(NBLK,),
    in_specs=[_row_spec(256), _full_spec(1, 256), _full_spec(1, 256),
              _full_spec(256, 128), _full_spec(1, 128)],
    out_specs=[_row_spec(128), _part_spec128, _part_spec128],
    out_shape=[
        jax.ShapeDtypeStruct((B, 128), jnp.float32),
        jax.ShapeDtypeStruct((NBLK, 1, 128), jnp.float32),
        jax.ShapeDtypeStruct((NBLK, 1, 128), jnp.float32),
    ],
)


def _phase3_body(h2_ref, sc_ref, sh_ref, w3_ref, b3_ref, w4_ref,
                 ue_ref, ee_ref, ub_ref, eb_ref, b4gb_ref, out_ref):
    h = h2_ref[...] * sc_ref[...] + sh_ref[...]
    h = jnp.maximum(jnp.dot(h, w3_ref[...], preferred_element_type=jnp.float32)
                    + b3_ref[...], 0.0)
    mlp_out = jnp.sum(h * w4_ref[...], axis=1, keepdims=True)
    mf = (jnp.sum(ue_ref[...] * ee_ref[...], axis=1, keepdims=True)
          + ub_ref[...] + eb_ref[...])
    out_ref[...] = jax.nn.sigmoid(mlp_out + 0.3 * mf + b4gb_ref[0, 0])


_phase3 = pl.pallas_call(
    _phase3_body,
    grid=(NBLK,),
    in_specs=[_row_spec(128), _full_spec(1, 128), _full_spec(1, 128),
              _full_spec(128, D), _full_spec(1, D), _full_spec(1, D),
              _row_spec(D), _row_spec(D), _row_spec(1), _row_spec(1),
              _full_spec(1, 1)],
    out_specs=_row_spec(1),
    out_shape=jax.ShapeDtypeStruct((B, 1), jnp.float32),
)


def _bn_coeffs(ps, pq, g, be):
    m = ps.sum(axis=0)[0] * (1.0 / B)
    v = pq.sum(axis=0)[0] * (1.0 / B) - m * m
    s = g * lax.rsqrt(v + _EPS)
    return s.reshape(1, -1), (be - m * s).reshape(1, -1)


def _mlp(ue, ee, ub, eb, w1a, w1b, b1, g1, be1, w2, b2, g2, be2,
         w3, b3, w4, b4gb):
    h1, ps1, pq1 = _phase1(ue, ee, w1a, w1b, b1.reshape(1, -1))
    sc1, sh1 = _bn_coeffs(ps1, pq1, g1, be1)
    h2, ps2, pq2 = _phase2(h1, sc1, sh1, w2, b2.reshape(1, -1))
    sc2, sh2 = _bn_coeffs(ps2, pq2, g2, be2)
    return _phase3(h2, sc2, sh2, w3, b3.reshape(1, -1), w4,
                   ue, ee, ub, eb, b4gb)


def kernel(user_ids, exercise_ids, user_emb, ex_emb, user_b, ex_b, global_b,
           W1, b1, g1, be1, W2, b2, g2, be2, W3, b3, W4, b4):
    uid = user_ids.astype(jnp.int32).reshape(NW, 1, BPW)
    eid = exercise_ids.astype(jnp.int32).reshape(NW, 1, BPW)
    ue, ee, ub, eb = _sc_gather()(uid, eid, user_emb, ex_emb,
                                  user_b.reshape(-1), ex_b.reshape(-1))
    w1a = W1[:, :D].T
    w1b = W1[:, D:].T
    b4gb = (0.7 * b4 + 0.3 * global_b).reshape(1, 1)
    return _mlp(ue, ee, ub.reshape(B, 1), eb.reshape(B, 1), w1a, w1b, b1,
                g1, be1, W2.T, b2, g2, be2, W3.T, b3,
                W4.reshape(1, D) * 0.7, b4gb)


# speedup vs baseline: 3.2506x; 1.4530x over previous
import functools

import jax
import jax.numpy as jnp
from jax import lax
from jax.experimental import pallas as pl
from jax.experimental.pallas import tpu as pltpu
from jax.experimental.pallas import tpu_sc as plsc

B = 16384
D = 64
NC = 2
NS = 16
NW = NC * NS
BPW = B // NW


def _sc_bias_body(uid_hbm, eid_hbm, ub_hbm, eb_hbm,
                  ub_out, eb_out,
                  uidx_v, eidx_v, ubv, ebv, semb):
    wid = lax.axis_index("s") * NC + lax.axis_index("c")
    base = wid * BPW
    pltpu.sync_copy(uid_hbm.at[wid], uidx_v)
    pltpu.sync_copy(eid_hbm.at[wid], eidx_v)
    cub = pltpu.async_copy(ub_hbm.at[uidx_v.at[0]], ubv, semb)
    ceb = pltpu.async_copy(eb_hbm.at[eidx_v.at[0]], ebv, semb)
    cub.wait()
    ceb.wait()
    pltpu.sync_copy(ubv, ub_out.at[pl.ds(base, BPW)])
    pltpu.sync_copy(ebv, eb_out.at[pl.ds(base, BPW)])


@functools.cache
def _sc_bias_gather():
    return pl.kernel(
        _sc_bias_body,
        out_type=[
            jax.ShapeDtypeStruct((B,), jnp.float32),
            jax.ShapeDtypeStruct((B,), jnp.float32),
        ],
        mesh=plsc.VectorSubcoreMesh(core_axis_name="c", subcore_axis_name="s"),
        compiler_params=pltpu.CompilerParams(use_tc_tiling_on_sc=False),
        scratch_types=[
            pltpu.VMEM((1, BPW), jnp.int32),
            pltpu.VMEM((1, BPW), jnp.int32),
            pltpu.VMEM((BPW,), jnp.float32),
            pltpu.VMEM((BPW,), jnp.float32),
            pltpu.SemaphoreType.DMA,
        ],
    )


RPC = BPW // 2


def _sc_emb_body(uid_hbm, eid_hbm, uemb_hbm, eemb_hbm,
                 ue_out, ee_out,
                 uidx_v, eidx_v, urows_v, erows_v, semu, seme):
    wid = lax.axis_index("s") * NC + lax.axis_index("c")
    base = wid * BPW
    pltpu.sync_copy(uid_hbm.at[wid], uidx_v)
    pltpu.sync_copy(eid_hbm.at[wid], eidx_v)
    for h in range(2):
        def issue(g, _):
            chunk_u = uidx_v[0, pl.ds(h * RPC + g * 16, 16)]
            chunk_e = eidx_v[0, pl.ds(h * RPC + g * 16, 16)]
            for l in range(16):
                iu = chunk_u[l]
                ie = chunk_e[l]
                r = g * 16 + l
                pltpu.async_copy(uemb_hbm.at[pl.ds(iu, 1)],
                                 urows_v.at[pl.ds(r, 1)], semu)
                pltpu.async_copy(eemb_hbm.at[pl.ds(ie, 1)],
                                 erows_v.at[pl.ds(r, 1)], seme)
            return 0

        lax.fori_loop(0, RPC // 16, issue, 0)
        pltpu.make_async_copy(uemb_hbm.at[pl.ds(0, RPC)], urows_v, semu).wait()
        pltpu.make_async_copy(eemb_hbm.at[pl.ds(0, RPC)], erows_v, seme).wait()
        pltpu.sync_copy(urows_v, ue_out.at[pl.ds(base + h * RPC, RPC)])
        pltpu.sync_copy(erows_v, ee_out.at[pl.ds(base + h * RPC, RPC)])


@functools.cache
def _sc_emb_gather():
    return pl.kernel(
        _sc_emb_body,
        out_type=[
            jax.ShapeDtypeStruct((B, D), jnp.float32),
            jax.ShapeDtypeStruct((B, D), jnp.float32),
        ],
        mesh=plsc.VectorSubcoreMesh(core_axis_name="c", subcore_axis_name="s"),
        scratch_types=[
            pltpu.VMEM((1, BPW), jnp.int32),
            pltpu.VMEM((1, BPW), jnp.int32),
            pltpu.VMEM((RPC, D), jnp.float32),
            pltpu.VMEM((RPC, D), jnp.float32),
            pltpu.SemaphoreType.DMA,
            pltpu.SemaphoreType.DMA,
        ],
    )


BLK = 1024
NBLK = B // BLK
_EPS = 1e-5

_row_spec = lambda w: pl.BlockSpec((BLK, w), lambda t: (t, 0))
_full_spec = lambda r, c: pl.BlockSpec((r, c), lambda t: (0, 0))
_part_spec = pl.BlockSpec((1, 1, 256), lambda t: (t, 0, 0))
_part_spec128 = pl.BlockSpec((1, 1, 128), lambda t: (t, 0, 0))


def _phase1_body(ue_ref, ee_ref, w1a_ref, w1b_ref, b1_ref,
                 h1_ref, ps_ref, pq_ref):
    h = (jnp.dot(ue_ref[...], w1a_ref[...], preferred_element_type=jnp.float32)
         + jnp.dot(ee_ref[...], w1b_ref[...], preferred_element_type=jnp.float32)
         + b1_ref[...])
    h = jnp.maximum(h, 0.0)
    h1_ref[...] = h
    ps_ref[...] = jnp.sum(h, axis=0, keepdims=True).reshape(1, 1, 256)
    pq_ref[...] = jnp.sum(h * h, axis=0, keepdims=True).reshape(1, 1, 256)


_phase1 = pl.pallas_call(
    _phase1_body,
    grid=(NBLK,),
    in_specs=[_row_spec(D), _row_spec(D), _full_spec(D, 256),
              _full_spec(D, 256), _full_spec(1, 256)],
    out_specs=[_row_spec(256), _part_spec, _part_spec],
    out_shape=[
        jax.ShapeDtypeStruct((B, 256), jnp.float32),
        jax.ShapeDtypeStruct((NBLK, 1, 256), jnp.float32),
        jax.ShapeDtypeStruct((NBLK, 1, 256), jnp.float32),
    ],
)


def _phase2_body(h1_ref, sc_ref, sh_ref, w2_ref, b2_ref,
                 h2_ref, ps_ref, pq_ref):
    h = h1_ref[...] * sc_ref[...] + sh_ref[...]
    h = jnp.maximum(jnp.dot(h, w2_ref[...], preferred_element_type=jnp.float32)
                    + b2_ref[...], 0.0)
    h2_ref[...] = h
    ps_ref[...] = jnp.sum(h, axis=0, keepdims=True).reshape(1, 1, 128)
    pq_ref[...] = jnp.sum(h * h, axis=0, keepdims=True).reshape(1, 1, 128)


_phase2 = pl.pallas_call(
    _phase2_body,
    grid=(NBLK,),
    in_specs=[_row_spec(256), _full_spec(1, 256), _full_spec(1, 256),
              _full_spec(256, 128), _full_spec(1, 128)],
    out_specs=[_row_spec(128), _part_spec128, _part_spec128],
    out_shape=[
        jax.ShapeDtypeStruct((B, 128), jnp.float32),
        jax.ShapeDtypeStruct((NBLK, 1, 128), jnp.float32),
        jax.ShapeDtypeStruct((NBLK, 1, 128), jnp.float32),
    ],
)


def _phase3_body(h2_ref, sc_ref, sh_ref, w3_ref, b3_ref, w4_ref,
                 ue_ref, ee_ref, ub_ref, eb_ref, b4gb_ref, out_ref):
    h = h2_ref[...] * sc_ref[...] + sh_ref[...]
    h = jnp.maximum(jnp.dot(h, w3_ref[...], preferred_element_type=jnp.float32)
                    + b3_ref[...], 0.0)
    mlp_out = jnp.sum(h * w4_ref[...], axis=1, keepdims=True)
    mf = (jnp.sum(ue_ref[...] * ee_ref[...], axis=1, keepdims=True)
          + ub_ref[...] + eb_ref[...])
    out_ref[...] = jax.nn.sigmoid(mlp_out + 0.3 * mf + b4gb_ref[0, 0])


_phase3 = pl.pallas_call(
    _phase3_body,
    grid=(NBLK,),
    in_specs=[_row_spec(128), _full_spec(1, 128), _full_spec(1, 128),
              _full_spec(128, D), _full_spec(1, D), _full_spec(1, D),
              _row_spec(D), _row_spec(D), _row_spec(1), _row_spec(1),
              _full_spec(1, 1)],
    out_specs=_row_spec(1),
    out_shape=jax.ShapeDtypeStruct((B, 1), jnp.float32),
)


def _bn_coeffs(ps, pq, g, be):
    m = ps.sum(axis=0)[0] * (1.0 / B)
    v = pq.sum(axis=0)[0] * (1.0 / B) - m * m
    s = g * lax.rsqrt(v + _EPS)
    return s.reshape(1, -1), (be - m * s).reshape(1, -1)


def _mlp(ue, ee, ub, eb, w1a, w1b, b1, g1, be1, w2, b2, g2, be2,
         w3, b3, w4, b4gb):
    h1, ps1, pq1 = _phase1(ue, ee, w1a, w1b, b1.reshape(1, -1))
    sc1, sh1 = _bn_coeffs(ps1, pq1, g1, be1)
    h2, ps2, pq2 = _phase2(h1, sc1, sh1, w2, b2.reshape(1, -1))
    sc2, sh2 = _bn_coeffs(ps2, pq2, g2, be2)
    return _phase3(h2, sc2, sh2, w3, b3.reshape(1, -1), w4,
                   ue, ee, ub, eb, b4gb)


def kernel(user_ids, exercise_ids, user_emb, ex_emb, user_b, ex_b, global_b,
           W1, b1, g1, be1, W2, b2, g2, be2, W3, b3, W4, b4):
    uid = user_ids.astype(jnp.int32).reshape(NW, 1, BPW)
    eid = exercise_ids.astype(jnp.int32).reshape(NW, 1, BPW)
    ue, ee = _sc_emb_gather()(uid, eid, user_emb, ex_emb)
    ub, eb = _sc_bias_gather()(uid, eid, user_b.reshape(-1),
                               ex_b.reshape(-1))
    w1a = W1[:, :D].T
    w1b = W1[:, D:].T
    b4gb = (0.7 * b4 + 0.3 * global_b).reshape(1, 1)
    return _mlp(ue, ee, ub.reshape(B, 1), eb.reshape(B, 1), w1a, w1b, b1,
                g1, be1, W2.T, b2, g2, be2, W3.T, b3,
                W4.reshape(1, D) * 0.7, b4gb)
